# baseline (speedup 1.0000x reference)
import functools

import jax
import jax.numpy as jnp
from jax.experimental import pallas as pl
from jax.experimental.pallas import tpu as pltpu

_TM = 256
_TN = 2048
_WCHUNK = 256
_NSTAGE = 2


def _linear_kernel(w_hbm, x_ref, b_ref, o_ref, wb_ref, stage_ref, sem):
    j = pl.program_id(0)
    t = pl.program_id(1)
    nchunks = _TN // _WCHUNK

    def copy(half, c, buf):
        return pltpu.make_async_copy(
            w_hbm.at[pl.ds(half * _TN + c * _WCHUNK, _WCHUNK), :],
            stage_ref.at[buf],
            sem.at[buf],
        )

    eye = (jax.lax.broadcasted_iota(jnp.int32, (_WCHUNK, _WCHUNK), 0)
           == jax.lax.broadcasted_iota(jnp.int32, (_WCHUNK, _WCHUNK), 1)
           ).astype(jnp.bfloat16)

    def cast(half, c, buf):
        chunk = stage_ref[buf].astype(jnp.bfloat16)
        wb_ref[half, :, pl.ds(c * _WCHUNK, _WCHUNK)] = jax.lax.dot_general(
            chunk, eye, (((0,), (0,)), ((), ())),
            preferred_element_type=jnp.float32).astype(jnp.bfloat16)

    @pl.when((j == 0) & (t == 0))
    def _():
        copy(0, 0, 0).start()
        for c in range(nchunks):
            if c + 1 < nchunks:
                copy(0, c + 1, (c + 1) % 2).start()
            copy(0, c, c % 2).wait()
            cast(0, c, c % 2)

    for c in range(nchunks):
        @pl.when((j == 0) & (t == c + 1))
        def _(c=c):
            copy(1, c, c % 2).start()

        @pl.when((j == 0) & (t == c + 2))
        def _(c=c):
            copy(1, c, c % 2).wait()
            cast(1, c, c % 2)

    xb = x_ref[...].astype(jnp.bfloat16)
    o_ref[...] = jnp.dot(
        xb, wb_ref[j], preferred_element_type=jnp.float32) + b_ref[...]


@functools.partial(jax.jit, static_argnames=())
def _linear(x2d, weight, b2):
    M, K = x2d.shape
    N = weight.shape[0]
    grid = (N // _TN, M // _TM)
    return pl.pallas_call(
        _linear_kernel,
        out_shape=jax.ShapeDtypeStruct((M, N), jnp.float32),
        grid=grid,
        in_specs=[
            pl.BlockSpec(memory_space=pl.ANY),
            pl.BlockSpec((_TM, K), lambda j, t: (t, 0)),
            pl.BlockSpec((1, _TN), lambda j, t: (0, j)),
        ],
        out_specs=pl.BlockSpec((_TM, _TN), lambda j, t: (t, j)),
        scratch_shapes=[
            pltpu.VMEM((2, K, _TN), jnp.bfloat16),
            pltpu.VMEM((_NSTAGE, _WCHUNK, K), jnp.float32),
            pltpu.SemaphoreType.DMA((_NSTAGE,)),
        ],
        compiler_params=pltpu.CompilerParams(
            dimension_semantics=("arbitrary", "arbitrary"),
            vmem_limit_bytes=100 * 1024 * 1024,
        ),
        cost_estimate=pl.CostEstimate(
            flops=2 * M * N * K,
            transcendentals=0,
            bytes_accessed=(M * K + N * K + M * N) * 4,
        ),
    )(weight, x2d, b2)


def kernel(x, weight, bias):
    orig_shape = x.shape
    K = orig_shape[-1]
    N = weight.shape[0]
    x2d = x.reshape(-1, K)
    out = _linear(x2d, weight, bias.reshape(1, N))
    return out.reshape(*orig_shape[:-1], N)


# speedup vs baseline: 1.0476x; 1.0476x over previous
import functools

import jax
import jax.numpy as jnp
from jax.experimental import pallas as pl
from jax.experimental.pallas import tpu as pltpu

_TM = 256
_TN = 2048
_WCHUNK = 256
_NSTAGE = 2


def _linear_kernel(w_hbm, x_ref, b_ref, o_ref, wb_ref, stage_ref, sem):
    j = pl.program_id(0)
    t = pl.program_id(1)
    nchunks = _TN // _WCHUNK

    def copy(half, c, buf):
        return pltpu.make_async_copy(
            w_hbm.at[pl.ds(half * _TN + c * _WCHUNK, _WCHUNK), :],
            stage_ref.at[buf],
            sem.at[buf],
        )

    def cast(half, c, buf):
        wb_ref[half, pl.ds(c * _WCHUNK, _WCHUNK), :] = (
            stage_ref[buf].astype(jnp.bfloat16))

    @pl.when((j == 0) & (t == 0))
    def _():
        copy(0, 0, 0).start()
        for c in range(nchunks):
            if c + 1 < nchunks:
                copy(0, c + 1, (c + 1) % 2).start()
            copy(0, c, c % 2).wait()
            cast(0, c, c % 2)

    for c in range(nchunks):
        @pl.when((j == 0) & (t == c + 1))
        def _(c=c):
            copy(1, c, c % 2).start()

        @pl.when((j == 0) & (t == c + 2))
        def _(c=c):
            copy(1, c, c % 2).wait()
            cast(1, c, c % 2)

    xb = x_ref[...].astype(jnp.bfloat16)
    o_ref[...] = jax.lax.dot_general(
        xb, wb_ref[j], (((1,), (1,)), ((), ())),
        preferred_element_type=jnp.float32) + b_ref[...]


@functools.partial(jax.jit, static_argnames=())
def _linear(x2d, weight, b2):
    M, K = x2d.shape
    N = weight.shape[0]
    grid = (N // _TN, M // _TM)
    return pl.pallas_call(
        _linear_kernel,
        out_shape=jax.ShapeDtypeStruct((M, N), jnp.float32),
        grid=grid,
        in_specs=[
            pl.BlockSpec(memory_space=pl.ANY),
            pl.BlockSpec((_TM, K), lambda j, t: (t, 0)),
            pl.BlockSpec((1, _TN), lambda j, t: (0, j)),
        ],
        out_specs=pl.BlockSpec((_TM, _TN), lambda j, t: (t, j)),
        scratch_shapes=[
            pltpu.VMEM((2, _TN, K), jnp.bfloat16),
            pltpu.VMEM((_NSTAGE, _WCHUNK, K), jnp.float32),
            pltpu.SemaphoreType.DMA((_NSTAGE,)),
        ],
        compiler_params=pltpu.CompilerParams(
            dimension_semantics=("arbitrary", "arbitrary"),
            vmem_limit_bytes=100 * 1024 * 1024,
        ),
        cost_estimate=pl.CostEstimate(
            flops=2 * M * N * K,
            transcendentals=0,
            bytes_accessed=(M * K + N * K + M * N) * 4,
        ),
    )(weight, x2d, b2)


def kernel(x, weight, bias):
    orig_shape = x.shape
    K = orig_shape[-1]
    N = weight.shape[0]
    x2d = x.reshape(-1, K)
    out = _linear(x2d, weight, bias.reshape(1, N))
    return out.reshape(*orig_shape[:-1], N)


# speedup vs baseline: 1.0540x; 1.0061x over previous
import functools

import jax
import jax.numpy as jnp
from jax.experimental import pallas as pl
from jax.experimental.pallas import tpu as pltpu

_TM = 256
_TN = 2048
_WCHUNK = 256
_NSTAGE = 2


def _linear_kernel(w_hbm, x_ref, b_ref, o_ref, wb_ref, stage_ref, sem):
    j = pl.program_id(0)
    t = pl.program_id(1)
    nchunks = _TN // _WCHUNK

    def copy(half, c, buf):
        return pltpu.make_async_copy(
            w_hbm.at[pl.ds(half * _TN + c * _WCHUNK, _WCHUNK), :],
            stage_ref.at[buf],
            sem.at[buf],
        )

    def cast(half, c, buf):
        wb_ref[half, pl.ds(c * _WCHUNK, _WCHUNK), :] = (
            stage_ref[buf].astype(jnp.bfloat16))

    first = (j == 0) & (t == 0)

    @pl.when(first)
    def _():
        xb0 = x_ref[...].astype(jnp.bfloat16)
        copy(0, 0, 0).start()
        for c in range(nchunks):
            if c + 1 < nchunks:
                copy(0, c + 1, (c + 1) % 2).start()
            copy(0, c, c % 2).wait()
            wc = stage_ref[c % 2].astype(jnp.bfloat16)
            wb_ref[0, pl.ds(c * _WCHUNK, _WCHUNK), :] = wc
            cols = pl.ds(c * _WCHUNK, _WCHUNK)
            o_ref[:, cols] = jax.lax.dot_general(
                xb0, wc, (((1,), (1,)), ((), ())),
                preferred_element_type=jnp.float32) + b_ref[:, cols]

    for c in range(nchunks):
        @pl.when((j == 0) & (t == c + 1))
        def _(c=c):
            copy(1, c, c % 2).start()

        @pl.when((j == 0) & (t == c + 2))
        def _(c=c):
            copy(1, c, c % 2).wait()
            cast(1, c, c % 2)

    @pl.when(jnp.logical_not(first))
    def _():
        xb = x_ref[...].astype(jnp.bfloat16)
        o_ref[...] = jax.lax.dot_general(
            xb, wb_ref[j], (((1,), (1,)), ((), ())),
            preferred_element_type=jnp.float32) + b_ref[...]


@functools.partial(jax.jit, static_argnames=())
def _linear(x2d, weight, b2):
    M, K = x2d.shape
    N = weight.shape[0]
    grid = (N // _TN, M // _TM)
    return pl.pallas_call(
        _linear_kernel,
        out_shape=jax.ShapeDtypeStruct((M, N), jnp.float32),
        grid=grid,
        in_specs=[
            pl.BlockSpec(memory_space=pl.ANY),
            pl.BlockSpec((_TM, K), lambda j, t: (t, 0)),
            pl.BlockSpec((1, _TN), lambda j, t: (0, j)),
        ],
        out_specs=pl.BlockSpec((_TM, _TN), lambda j, t: (t, j)),
        scratch_shapes=[
            pltpu.VMEM((2, _TN, K), jnp.bfloat16),
            pltpu.VMEM((_NSTAGE, _WCHUNK, K), jnp.float32),
            pltpu.SemaphoreType.DMA((_NSTAGE,)),
        ],
        compiler_params=pltpu.CompilerParams(
            dimension_semantics=("arbitrary", "arbitrary"),
            vmem_limit_bytes=100 * 1024 * 1024,
        ),
        cost_estimate=pl.CostEstimate(
            flops=2 * M * N * K,
            transcendentals=0,
            bytes_accessed=(M * K + N * K + M * N) * 4,
        ),
    )(weight, x2d, b2)


def kernel(x, weight, bias):
    orig_shape = x.shape
    K = orig_shape[-1]
    N = weight.shape[0]
    x2d = x.reshape(-1, K)
    out = _linear(x2d, weight, bias.reshape(1, N))
    return out.reshape(*orig_shape[:-1], N)
